# baseline (device time: 64939 ns/iter reference)
import functools

import jax
import jax.numpy as jnp
import numpy as np
from jax import lax
from jax.experimental import pallas as pl
from jax.experimental.pallas import tpu as pltpu

N_DEV = 32
SUB = 4

L_R = (16, 16, 15, 15)
O_R = (0, 1, 2, 3)
L_L = (15, 15, 16, 16)
O_L = (2, 3, 0, 1)
N_SLOTS = max(max(l - 1 + o for l, o in zip(L_R, O_R)),
              max(l - 1 + o for l, o in zip(L_L, O_L))) + 1
COMB_SLOT = tuple(
    max(L_R[k] - 1 + O_R[k], L_L[k] - 1 + O_L[k]) for k in range(SUB)
)
MAX_L = max(max(L_R), max(L_L))


def _cycle_tables():
    identity = np.arange(N_DEV)
    ring = (identity, (identity + 1) % N_DEV, (identity - 1) % N_DEV, identity)
    try:
        import distributed_mesh_v7x as dm

        mesh = dm.get_mesh("i", world_size=N_DEV)
        coords = [tuple(d.coords) for d in mesh.devices.flat]
    except Exception:
        return ring
    xs = sorted({c[0] for c in coords})
    ys = sorted({c[1] for c in coords})
    zs = sorted({c[2] for c in coords})
    if len(coords) != N_DEV or len(xs) != 2 or len(ys) != 4 or len(zs) != 4:
        return ring
    snake = [
        (y, z)
        for y in ys
        for z in (zs if y % 2 == 0 else list(reversed(zs)))
    ]
    cycle = [(xs[0], y, z) for (y, z) in snake] + [
        (xs[1], y, z) for (y, z) in reversed(snake)
    ]
    if set(cycle) != set(coords):
        return ring
    if any(
        sum(abs(a - b) for a, b in zip(cycle[i], cycle[(i + 1) % N_DEV])) != 1
        for i in range(N_DEV)
    ):
        return ring
    log_of = {c: i for i, c in enumerate(coords)}
    ldev = np.array([log_of[c] for c in cycle], dtype=np.int32)
    pos = np.empty(N_DEV, dtype=np.int32)
    pos[ldev] = np.arange(N_DEV, dtype=np.int32)
    nxt = np.empty(N_DEV, dtype=np.int32)
    prv = np.empty(N_DEV, dtype=np.int32)
    for p in range(N_DEV):
        nxt[ldev[p]] = ldev[(p + 1) % N_DEV]
        prv[ldev[p]] = ldev[(p - 1) % N_DEV]
    return pos, nxt, prv, ldev


def kernel(x, w_mat):
    k_dim, m_loc = x.shape
    _, n = w_mat.shape
    m_glob = k_dim
    m_per = m_glob // N_DEV
    w_sub = n // SUB

    pos_t, nxt_t, prv_t, ldev_t = (jnp.asarray(t, jnp.int32) for t in _cycle_tables())

    def body(pos_ref, nxt_ref, prv_ref, ldev_ref, x_ref, w_ref, out_ref,
             p_ref, r_comm, l_comm, r_send, r_recv, l_send, l_recv):
        d = lax.axis_index("i")
        q = pos_ref[d]
        right = nxt_ref[d]
        left = prv_ref[d]

        barrier_sem = pltpu.get_barrier_semaphore()
        for nbr in (left, right):
            pl.semaphore_signal(
                barrier_sem, inc=1,
                device_id=(nbr,), device_id_type=pl.DeviceIdType.MESH,
            )
        pl.semaphore_wait(barrier_sem, 2)

        def pchunk(c, k):
            return p_ref[pl.ds(c * m_per, m_per), k * w_sub:(k + 1) * w_sub]

        def mk(comm, sems_send, sems_recv, s, k, dev):
            return pltpu.make_async_remote_copy(
                src_ref=comm.at[s, k],
                dst_ref=comm.at[s + 1, k],
                send_sem=sems_send.at[s, k],
                recv_sem=sems_recv.at[s, k],
                device_id=(dev,),
                device_id_type=pl.DeviceIdType.MESH,
            )

        for k in range(SUB):
            cols = slice(k * w_sub, (k + 1) * w_sub)
            cr_k = ldev_ref[(q + L_R[k]) % N_DEV]
            cl_k = ldev_ref[(q - L_L[k]) % N_DEV]
            r_comm[0, k, :, :] = jnp.dot(
                x_ref[pl.ds(cr_k * m_per, m_per), :], w_ref[:, cols],
                preferred_element_type=jnp.float32,
            ).astype(jnp.bfloat16)
            l_comm[0, k, :, :] = jnp.dot(
                x_ref[pl.ds(cl_k * m_per, m_per), :], w_ref[:, cols],
                preferred_element_type=jnp.float32,
            ).astype(jnp.bfloat16)
            mk(r_comm, r_send, r_recv, 0, k, right).start()
            mk(l_comm, l_send, l_recv, 0, k, left).start()

        p_ref[...] = jnp.dot(
            x_ref[...], w_ref[...], preferred_element_type=jnp.float32
        ).astype(jnp.bfloat16)

        def slot(t, carry):
            for k in range(SUB):
                s = t - O_R[k]
                lr = L_R[k]
                act = jnp.logical_and(s >= 0, s < lr)
                sc = jnp.clip(s, 0, lr - 1)

                @pl.when(act)
                def _():
                    mk(r_comm, r_send, r_recv, sc, k, right).wait_recv()

                @pl.when(jnp.logical_and(act, s < lr - 1))
                def _():
                    c = 0
                    r_comm[sc + 1, k, :, :] = r_comm[sc + 1, k, :, :] + pchunk(c, k)
                    mk(r_comm, r_send, r_recv, sc + 1, k, right).start()

            for k in sorted(range(SUB), key=lambda kk: O_L[kk]):
                s = t - O_L[k]
                ll = L_L[k]
                act = jnp.logical_and(s >= 0, s < ll)
                sc = jnp.clip(s, 0, ll - 1)

                @pl.when(act)
                def _():
                    mk(l_comm, l_send, l_recv, sc, k, left).wait_recv()

                @pl.when(jnp.logical_and(act, s < ll - 1))
                def _():
                    c = 0
                    l_comm[sc + 1, k, :, :] = l_comm[sc + 1, k, :, :] + pchunk(c, k)
                    mk(l_comm, l_send, l_recv, sc + 1, k, left).start()

            for k in range(SUB):

                @pl.when(t == COMB_SLOT[k])
                def _():
                    total = (
                        pchunk(d, k).astype(jnp.float32)
                        + r_comm[L_R[k], k, :, :].astype(jnp.float32)
                        + l_comm[L_L[k], k, :, :].astype(jnp.float32)
                    )
                    out_ref[:, k * w_sub:(k + 1) * w_sub] = jnp.maximum(total, 0.0)

            return carry

        lax.fori_loop(0, N_SLOTS, slot, 0)

        def drain(t, carry):
            for k in range(SUB):

                @pl.when(t < L_R[k])
                def _():
                    mk(r_comm, r_send, r_recv, jnp.clip(t, 0, L_R[k] - 1), k,
                       right).wait_send()

                @pl.when(t < L_L[k])
                def _():
                    mk(l_comm, l_send, l_recv, jnp.clip(t, 0, L_L[k] - 1), k,
                       left).wait_send()

            return carry

        lax.fori_loop(0, MAX_L, drain, 0)

        @functools.partial(
            pl.run_scoped, second_barrier=pltpu.SemaphoreType.REGULAR
        )
        def _(second_barrier):
            for nbr in (left, right):
                pl.semaphore_signal(
                    second_barrier, inc=1,
                    device_id=(nbr,), device_id_type=pl.DeviceIdType.MESH,
                )
            pl.semaphore_wait(second_barrier, 2)

    return pl.pallas_call(
        body,
        out_shape=jax.ShapeDtypeStruct((m_per, n), jnp.float32),
        in_specs=[
            pl.BlockSpec(memory_space=pltpu.SMEM),
            pl.BlockSpec(memory_space=pltpu.SMEM),
            pl.BlockSpec(memory_space=pltpu.SMEM),
            pl.BlockSpec(memory_space=pltpu.SMEM),
            pl.BlockSpec(memory_space=pltpu.VMEM),
            pl.BlockSpec(memory_space=pltpu.VMEM),
        ],
        out_specs=pl.BlockSpec(memory_space=pltpu.VMEM),
        scratch_shapes=[
            pltpu.VMEM((m_glob, n), jnp.bfloat16),
            pltpu.VMEM((MAX_L + 1, SUB, m_per, w_sub), jnp.bfloat16),
            pltpu.VMEM((MAX_L + 1, SUB, m_per, w_sub), jnp.bfloat16),
            pltpu.SemaphoreType.DMA((MAX_L, SUB)),
            pltpu.SemaphoreType.DMA((MAX_L, SUB)),
            pltpu.SemaphoreType.DMA((MAX_L, SUB)),
            pltpu.SemaphoreType.DMA((MAX_L, SUB)),
        ],
        compiler_params=pltpu.CompilerParams(collective_id=0),
    )(pos_t, nxt_t, prv_t, ldev_t, x, w_mat)


# device time: 64740 ns/iter; 1.0031x vs baseline; 1.0031x over previous
import functools

import jax
import jax.numpy as jnp
import numpy as np
from jax import lax
from jax.experimental import pallas as pl
from jax.experimental.pallas import tpu as pltpu

N_DEV = 32
SUB = 4

L_R = (16, 16, 15, 15)
O_R = (0, 1, 2, 3)
L_L = (15, 15, 16, 16)
O_L = (2, 3, 0, 1)
N_SLOTS = max(max(l - 1 + o for l, o in zip(L_R, O_R)),
              max(l - 1 + o for l, o in zip(L_L, O_L))) + 1
COMB_SLOT = tuple(
    max(L_R[k] - 1 + O_R[k], L_L[k] - 1 + O_L[k]) for k in range(SUB)
)
MAX_L = max(max(L_R), max(L_L))


def _cycle_tables():
    identity = np.arange(N_DEV)
    ring = (identity, (identity + 1) % N_DEV, (identity - 1) % N_DEV, identity)
    try:
        import distributed_mesh_v7x as dm

        mesh = dm.get_mesh("i", world_size=N_DEV)
        coords = [tuple(d.coords) for d in mesh.devices.flat]
    except Exception:
        return ring
    xs = sorted({c[0] for c in coords})
    ys = sorted({c[1] for c in coords})
    zs = sorted({c[2] for c in coords})
    if len(coords) != N_DEV or len(xs) != 2 or len(ys) != 4 or len(zs) != 4:
        return ring
    snake = [
        (y, z)
        for y in ys
        for z in (zs if y % 2 == 0 else list(reversed(zs)))
    ]
    cycle = [(xs[0], y, z) for (y, z) in snake] + [
        (xs[1], y, z) for (y, z) in reversed(snake)
    ]
    if set(cycle) != set(coords):
        return ring
    if any(
        sum(abs(a - b) for a, b in zip(cycle[i], cycle[(i + 1) % N_DEV])) != 1
        for i in range(N_DEV)
    ):
        return ring
    log_of = {c: i for i, c in enumerate(coords)}
    ldev = np.array([log_of[c] for c in cycle], dtype=np.int32)
    pos = np.empty(N_DEV, dtype=np.int32)
    pos[ldev] = np.arange(N_DEV, dtype=np.int32)
    nxt = np.empty(N_DEV, dtype=np.int32)
    prv = np.empty(N_DEV, dtype=np.int32)
    for p in range(N_DEV):
        nxt[ldev[p]] = ldev[(p + 1) % N_DEV]
        prv[ldev[p]] = ldev[(p - 1) % N_DEV]
    return pos, nxt, prv, ldev


def kernel(x, w_mat):
    k_dim, m_loc = x.shape
    _, n = w_mat.shape
    m_glob = k_dim
    m_per = m_glob // N_DEV
    w_sub = n // SUB

    pos_t, nxt_t, prv_t, ldev_t = (jnp.asarray(t, jnp.int32) for t in _cycle_tables())

    def body(pos_ref, nxt_ref, prv_ref, ldev_ref, x_ref, w_ref, out_ref,
             p_ref, r_comm, l_comm, r_send, r_recv, l_send, l_recv):
        d = lax.axis_index("i")
        q = pos_ref[d]
        right = nxt_ref[d]
        left = prv_ref[d]

        barrier_sem = pltpu.get_barrier_semaphore()
        for nbr in (left, right):
            pl.semaphore_signal(
                barrier_sem, inc=1,
                device_id=(nbr,), device_id_type=pl.DeviceIdType.MESH,
            )
        pl.semaphore_wait(barrier_sem, 2)

        def pchunk(c, k):
            return p_ref[pl.ds(c * m_per, m_per), k * w_sub:(k + 1) * w_sub]

        def mk(comm, sems_send, sems_recv, s, k, dev):
            return pltpu.make_async_remote_copy(
                src_ref=comm.at[s, k],
                dst_ref=comm.at[s + 1, k],
                send_sem=sems_send.at[s, k],
                recv_sem=sems_recv.at[s, k],
                device_id=(dev,),
                device_id_type=pl.DeviceIdType.MESH,
            )

        for k in range(SUB):
            cols = slice(k * w_sub, (k + 1) * w_sub)
            cr_k = ldev_ref[(q + L_R[k]) % N_DEV]
            cl_k = ldev_ref[(q - L_L[k]) % N_DEV]
            r_comm[0, k, :, :] = jnp.dot(
                x_ref[pl.ds(cr_k * m_per, m_per), :], w_ref[:, cols],
                preferred_element_type=jnp.float32,
            ).astype(jnp.bfloat16)
            l_comm[0, k, :, :] = jnp.dot(
                x_ref[pl.ds(cl_k * m_per, m_per), :], w_ref[:, cols],
                preferred_element_type=jnp.float32,
            ).astype(jnp.bfloat16)
            mk(r_comm, r_send, r_recv, 0, k, right).start()
            mk(l_comm, l_send, l_recv, 0, k, left).start()

        p_ref[...] = jnp.dot(
            x_ref[...], w_ref[...], preferred_element_type=jnp.float32
        ).astype(jnp.bfloat16)

        for t in range(N_SLOTS):
            for k in range(SUB):
                s = t - O_R[k]
                lr = L_R[k]
                if 0 <= s < lr:
                    mk(r_comm, r_send, r_recv, s, k, right).wait_recv()
                    if s < lr - 1:
                        c = ldev_ref[(q + lr - 1 - s) % N_DEV]
                        r_comm[s + 1, k, :, :] = (
                            r_comm[s + 1, k, :, :] + pchunk(c, k)
                        )
                        mk(r_comm, r_send, r_recv, s + 1, k, right).start()

            for k in sorted(range(SUB), key=lambda kk: O_L[kk]):
                s = t - O_L[k]
                ll = L_L[k]
                if 0 <= s < ll:
                    mk(l_comm, l_send, l_recv, s, k, left).wait_recv()
                    if s < ll - 1:
                        c = ldev_ref[(q - ll + 1 + s) % N_DEV]
                        l_comm[s + 1, k, :, :] = (
                            l_comm[s + 1, k, :, :] + pchunk(c, k)
                        )
                        mk(l_comm, l_send, l_recv, s + 1, k, left).start()

            for k in range(SUB):
                if t == COMB_SLOT[k]:
                    total = (
                        pchunk(d, k).astype(jnp.float32)
                        + r_comm[L_R[k], k, :, :].astype(jnp.float32)
                        + l_comm[L_L[k], k, :, :].astype(jnp.float32)
                    )
                    out_ref[:, k * w_sub:(k + 1) * w_sub] = jnp.maximum(total, 0.0)

        for t in range(MAX_L):
            for k in range(SUB):
                if t < L_R[k]:
                    mk(r_comm, r_send, r_recv, t, k, right).wait_send()
                if t < L_L[k]:
                    mk(l_comm, l_send, l_recv, t, k, left).wait_send()

        @functools.partial(
            pl.run_scoped, second_barrier=pltpu.SemaphoreType.REGULAR
        )
        def _(second_barrier):
            for nbr in (left, right):
                pl.semaphore_signal(
                    second_barrier, inc=1,
                    device_id=(nbr,), device_id_type=pl.DeviceIdType.MESH,
                )
            pl.semaphore_wait(second_barrier, 2)

    return pl.pallas_call(
        body,
        out_shape=jax.ShapeDtypeStruct((m_per, n), jnp.float32),
        in_specs=[
            pl.BlockSpec(memory_space=pltpu.SMEM),
            pl.BlockSpec(memory_space=pltpu.SMEM),
            pl.BlockSpec(memory_space=pltpu.SMEM),
            pl.BlockSpec(memory_space=pltpu.SMEM),
            pl.BlockSpec(memory_space=pltpu.VMEM),
            pl.BlockSpec(memory_space=pltpu.VMEM),
        ],
        out_specs=pl.BlockSpec(memory_space=pltpu.VMEM),
        scratch_shapes=[
            pltpu.VMEM((m_glob, n), jnp.bfloat16),
            pltpu.VMEM((MAX_L + 1, SUB, m_per, w_sub), jnp.bfloat16),
            pltpu.VMEM((MAX_L + 1, SUB, m_per, w_sub), jnp.bfloat16),
            pltpu.SemaphoreType.DMA((MAX_L, SUB)),
            pltpu.SemaphoreType.DMA((MAX_L, SUB)),
            pltpu.SemaphoreType.DMA((MAX_L, SUB)),
            pltpu.SemaphoreType.DMA((MAX_L, SUB)),
        ],
        compiler_params=pltpu.CompilerParams(collective_id=0),
    )(pos_t, nxt_t, prv_t, ldev_t, x, w_mat)
